# Initial kernel scaffold; baseline (speedup 1.0000x reference)
import functools

import jax
import jax.numpy as jnp
from jax import lax
from jax.experimental import pallas as pl
from jax.experimental.pallas import tpu as pltpu
from jax.experimental.pallas import tpu_sc as plsc

N = 10000
E = 320000
D = 128
DE = 16
EPS = 0.0

NC = 2
NS = 16
L = 16
NW = NC * NS
EPT = E // NW
C = 80
NCHUNK = EPT // C
RPT = N // NS

_sc_mesh = plsc.VectorSubcoreMesh(
    core_axis_name="c", subcore_axis_name="s", num_cores=NC, num_subcores=NS
)


_BE = 2000


def _edge_mlp_body(x_ref, w1_ref, b1_ref, w2_ref, b2_ref, o_ref):
    x = x_ref[...]
    t = jnp.maximum(
        jnp.dot(x, w1_ref[...], preferred_element_type=jnp.float32) + b1_ref[...], 0.0
    )
    o_ref[...] = (
        jnp.dot(t, w2_ref[...], preferred_element_type=jnp.float32) + b2_ref[...]
    )


def _edge_mlp(ef, w1, b1, w2, b2):
    return pl.pallas_call(
        _edge_mlp_body,
        grid=(E // _BE,),
        in_specs=[
            pl.BlockSpec((_BE, DE), lambda i: (i, 0)),
            pl.BlockSpec((DE, D), lambda i: (0, 0)),
            pl.BlockSpec((1, D), lambda i: (0, 0)),
            pl.BlockSpec((D, D), lambda i: (0, 0)),
            pl.BlockSpec((1, D), lambda i: (0, 0)),
        ],
        out_specs=pl.BlockSpec((_BE, D), lambda i: (i, 0)),
        out_shape=jax.ShapeDtypeStruct((E, D), jnp.float32),
    )(ef, w1, b1, w2, b2)


@functools.partial(
    pl.kernel,
    out_type=jax.ShapeDtypeStruct((NC, N, D), jnp.float32),
    mesh=_sc_mesh,
    scratch_types=[
        pltpu.VMEM_SHARED((N, D), jnp.float32),
        pltpu.VMEM((C, D), jnp.float32),
        pltpu.VMEM((C, D), jnp.float32),
        pltpu.VMEM((C,), jnp.int32),
        pltpu.VMEM((C,), jnp.int32),
        pltpu.SemaphoreType.DMA,
    ],
)
def _sc_pass(e_hbm, h_hbm, src_hbm, dst_hbm, out_hbm, aggr, ebuf, hbuf, sidx, didx, sem):
    c = lax.axis_index("c")
    s = lax.axis_index("s")
    wid = s * NC + c

    def _zrow(i, carry):
        for j in range(D // L):
            ebuf[i, pl.ds(j * L, L)] = jnp.zeros((L,), jnp.float32)
        return carry

    lax.fori_loop(0, C, _zrow, 0)
    zbase = s * RPT
    for k in range(RPT // C):
        pltpu.sync_copy(ebuf, aggr.at[pl.ds(zbase + k * C, C)])
    rem = RPT % C
    if rem:
        pltpu.sync_copy(
            ebuf.at[pl.ds(0, rem)], aggr.at[pl.ds(zbase + RPT - rem, rem)]
        )
    plsc.subcore_barrier()

    tbase = wid * EPT

    def _chunk(g, carry):
        off = tbase + g * C
        pltpu.sync_copy(src_hbm.at[pl.ds(off, C)], sidx)
        pltpu.sync_copy(dst_hbm.at[pl.ds(off, C)], didx)
        pltpu.sync_copy(e_hbm.at[pl.ds(off, C)], ebuf)
        pltpu.async_copy(h_hbm.at[sidx], hbuf, sem).wait()

        def _row(i, inner):
            for j in range(D // L):
                sl = pl.ds(j * L, L)
                ebuf[i, sl] = jnp.maximum(hbuf[i, sl] + ebuf[i, sl], 0.0)
            return inner

        lax.fori_loop(0, C, _row, 0)
        pltpu.sync_copy(ebuf, aggr.at[didx], add=True)
        return carry

    lax.fori_loop(0, NCHUNK, _chunk, 0)
    plsc.subcore_barrier()

    pltpu.sync_copy(
        aggr.at[pl.ds(zbase, RPT)], out_hbm.at[c, pl.ds(zbase, RPT), :]
    )


_BN = 1000


def _apply_body(h_ref, p_ref, w_ref, b_ref, o_ref):
    x = (1.0 + EPS) * h_ref[...] + p_ref[0] + p_ref[1]
    o_ref[...] = jnp.maximum(
        jnp.dot(x, w_ref[...], preferred_element_type=jnp.float32) + b_ref[...], 0.0
    )


def _apply(h, partials, w, b):
    return pl.pallas_call(
        _apply_body,
        grid=(N // _BN,),
        in_specs=[
            pl.BlockSpec((_BN, D), lambda i: (i, 0)),
            pl.BlockSpec((NC, _BN, D), lambda i: (0, i, 0)),
            pl.BlockSpec((D, D), lambda i: (0, 0)),
            pl.BlockSpec((1, D), lambda i: (0, 0)),
        ],
        out_specs=pl.BlockSpec((_BN, D), lambda i: (i, 0)),
        out_shape=jax.ShapeDtypeStruct((N, D), jnp.float32),
    )(h, partials, w, b)


def kernel(node_feats, edge_feats, edge_index, We1, be1, We2, be2, W0, b0, W1, b1):
    src = edge_index[0]
    dst = edge_index[1]
    e = _edge_mlp(edge_feats, We1, be1.reshape(1, D), We2, be2.reshape(1, D))
    h = node_feats
    for w, b in ((W0, b0), (W1, b1)):
        partials = _sc_pass(e, h, src, dst)
        h = _apply(h, partials, w, b.reshape(1, D))
    return h


# speedup vs baseline: 2.5265x; 2.5265x over previous
import functools

import jax
import jax.numpy as jnp
from jax import lax
from jax.experimental import pallas as pl
from jax.experimental.pallas import tpu as pltpu
from jax.experimental.pallas import tpu_sc as plsc

N = 10000
E = 320000
D = 128
DE = 16
EPS = 0.0

NC = 2
NS = 16
L = 16
NW = NC * NS
EPT = E // NW
C = 80
NCHUNK = EPT // C
NP = 10240
RPT = NP // NS

_sc_mesh = plsc.VectorSubcoreMesh(
    core_axis_name="c", subcore_axis_name="s", num_cores=NC, num_subcores=NS
)


_BE = 2000


def _edge_mlp_body(x_ref, w1_ref, b1_ref, w2_ref, b2_ref, o_ref):
    x = x_ref[...]
    t = jnp.maximum(
        jnp.dot(x, w1_ref[...], preferred_element_type=jnp.float32) + b1_ref[...], 0.0
    )
    o_ref[...] = (
        jnp.dot(t, w2_ref[...], preferred_element_type=jnp.float32) + b2_ref[...]
    )


def _edge_mlp(ef, w1, b1, w2, b2):
    return pl.pallas_call(
        _edge_mlp_body,
        grid=(E // _BE,),
        in_specs=[
            pl.BlockSpec((_BE, DE), lambda i: (i, 0)),
            pl.BlockSpec((DE, D), lambda i: (0, 0)),
            pl.BlockSpec((1, D), lambda i: (0, 0)),
            pl.BlockSpec((D, D), lambda i: (0, 0)),
            pl.BlockSpec((1, D), lambda i: (0, 0)),
        ],
        out_specs=pl.BlockSpec((_BE, D), lambda i: (i, 0)),
        out_shape=jax.ShapeDtypeStruct((E, D), jnp.float32),
    )(ef, w1, b1, w2, b2)


@functools.partial(
    pl.kernel,
    out_type=jax.ShapeDtypeStruct((NC, NP, D), jnp.float32),
    mesh=_sc_mesh,
    scratch_types=[
        pltpu.VMEM_SHARED((NP, D), jnp.float32),
        pltpu.VMEM((C, D), jnp.float32),
        pltpu.VMEM((C, D), jnp.float32),
        pltpu.VMEM((C,), jnp.int32),
        pltpu.VMEM((C,), jnp.int32),
        pltpu.SemaphoreType.DMA,
    ],
)
def _sc_pass(e_hbm, h_hbm, src_hbm, dst_hbm, out_hbm, aggr, ebuf, hbuf, sidx, didx, sem):
    c = lax.axis_index("c")
    s = lax.axis_index("s")
    wid = s * NC + c

    def _zrow(i, carry):
        for j in range(D // L):
            ebuf[i, pl.ds(j * L, L)] = jnp.zeros((L,), jnp.float32)
        return carry

    lax.fori_loop(0, C, _zrow, 0)
    zbase = s * RPT
    for k in range(RPT // C):
        pltpu.sync_copy(ebuf, aggr.at[pl.ds(zbase + k * C, C)])
    plsc.subcore_barrier()

    tbase = wid * EPT

    def _chunk(g, carry):
        off = tbase + g * C
        pltpu.sync_copy(src_hbm.at[pl.ds(off, C)], sidx)
        pltpu.sync_copy(dst_hbm.at[pl.ds(off, C)], didx)
        pltpu.sync_copy(e_hbm.at[pl.ds(off, C)], ebuf)
        pltpu.async_copy(h_hbm.at[sidx], hbuf, sem).wait()

        def _row(i, inner):
            for j in range(D // L):
                sl = pl.ds(j * L, L)
                ebuf[i, sl] = jnp.maximum(hbuf[i, sl] + ebuf[i, sl], 0.0)
            return inner

        lax.fori_loop(0, C, _row, 0)
        pltpu.sync_copy(ebuf, aggr.at[didx], add=True)
        return carry

    lax.fori_loop(0, NCHUNK, _chunk, 0)
    plsc.subcore_barrier()

    pltpu.sync_copy(
        aggr.at[pl.ds(zbase, RPT)], out_hbm.at[c, pl.ds(zbase, RPT), :]
    )


_BN = 1000


def _apply_body(h_ref, p_ref, w_ref, b_ref, o_ref):
    x = (1.0 + EPS) * h_ref[...] + p_ref[0] + p_ref[1]
    o_ref[...] = jnp.maximum(
        jnp.dot(x, w_ref[...], preferred_element_type=jnp.float32) + b_ref[...], 0.0
    )


def _apply(h, partials, w, b):
    return pl.pallas_call(
        _apply_body,
        grid=(N // _BN,),
        in_specs=[
            pl.BlockSpec((_BN, D), lambda i: (i, 0)),
            pl.BlockSpec((NC, _BN, D), lambda i: (0, i, 0)),
            pl.BlockSpec((D, D), lambda i: (0, 0)),
            pl.BlockSpec((1, D), lambda i: (0, 0)),
        ],
        out_specs=pl.BlockSpec((_BN, D), lambda i: (i, 0)),
        out_shape=jax.ShapeDtypeStruct((N, D), jnp.float32),
    )(h, partials, w, b)


def kernel(node_feats, edge_feats, edge_index, We1, be1, We2, be2, W0, b0, W1, b1):
    src = edge_index[0]
    dst = edge_index[1]
    e = _edge_mlp(edge_feats, We1, be1.reshape(1, D), We2, be2.reshape(1, D))
    h = node_feats
    for w, b in ((W0, b0), (W1, b1)):
        partials = _sc_pass(e, h, src, dst)
        h = _apply(h, partials, w, b.reshape(1, D))
    return h


# speedup vs baseline: 4.4958x; 1.7795x over previous
import functools

import jax
import jax.numpy as jnp
from jax import lax
from jax.experimental import pallas as pl
from jax.experimental.pallas import tpu as pltpu
from jax.experimental.pallas import tpu_sc as plsc

N = 10000
E = 320000
D = 128
DE = 16
EPS = 0.0

NC = 2
NS = 16
L = 16
NW = NC * NS
EPT = E // NW
C = 40
NCHUNK = EPT // C
NP = 10240
RPT = NP // NS

_sc_mesh = plsc.VectorSubcoreMesh(
    core_axis_name="c", subcore_axis_name="s", num_cores=NC, num_subcores=NS
)


_BE = 2000


def _edge_mlp_body(x_ref, w1_ref, b1_ref, w2_ref, b2_ref, o_ref):
    x = x_ref[...]
    t = jnp.maximum(
        jnp.dot(x, w1_ref[...], preferred_element_type=jnp.float32) + b1_ref[...], 0.0
    )
    o_ref[...] = (
        jnp.dot(t, w2_ref[...], preferred_element_type=jnp.float32) + b2_ref[...]
    )


def _edge_mlp(ef, w1, b1, w2, b2):
    return pl.pallas_call(
        _edge_mlp_body,
        grid=(E // _BE,),
        in_specs=[
            pl.BlockSpec((_BE, DE), lambda i: (i, 0)),
            pl.BlockSpec((DE, D), lambda i: (0, 0)),
            pl.BlockSpec((1, D), lambda i: (0, 0)),
            pl.BlockSpec((D, D), lambda i: (0, 0)),
            pl.BlockSpec((1, D), lambda i: (0, 0)),
        ],
        out_specs=pl.BlockSpec((_BE, D), lambda i: (i, 0)),
        out_shape=jax.ShapeDtypeStruct((E, D), jnp.float32),
    )(ef, w1, b1, w2, b2)


@functools.partial(
    pl.kernel,
    out_type=jax.ShapeDtypeStruct((NC, NP, D), jnp.float32),
    mesh=_sc_mesh,
    scratch_types=[
        pltpu.VMEM_SHARED((NP, D), jnp.float32),
        pltpu.VMEM((C, D), jnp.float32),
        pltpu.VMEM((C, D), jnp.float32),
        pltpu.VMEM((C, D), jnp.float32),
        pltpu.VMEM((C, D), jnp.float32),
        pltpu.VMEM((C,), jnp.int32),
        pltpu.VMEM((C,), jnp.int32),
        pltpu.VMEM((C,), jnp.int32),
        pltpu.VMEM((C,), jnp.int32),
        pltpu.SemaphoreType.DMA,
        pltpu.SemaphoreType.DMA,
        pltpu.SemaphoreType.DMA,
        pltpu.SemaphoreType.DMA,
        pltpu.SemaphoreType.DMA,
        pltpu.SemaphoreType.DMA,
        pltpu.SemaphoreType.DMA,
        pltpu.SemaphoreType.DMA,
        pltpu.SemaphoreType.DMA,
        pltpu.SemaphoreType.DMA,
    ],
)
def _sc_pass(
    e_hbm, h_hbm, src_hbm, dst_hbm, out_hbm,
    aggr, ebuf0, ebuf1, hbuf0, hbuf1, sibuf0, sibuf1, dbuf0, dbuf1,
    seme0, seme1, semg0, semg1, sems0, sems1, semi0, semi1, semd0, semd1,
):
    c = lax.axis_index("c")
    s = lax.axis_index("s")
    wid = s * NC + c

    ebufs = (ebuf0, ebuf1)
    hbufs = (hbuf0, hbuf1)
    sibufs = (sibuf0, sibuf1)
    dbufs = (dbuf0, dbuf1)
    semes = (seme0, seme1)
    semgs = (semg0, semg1)
    semss = (sems0, sems1)
    semis = (semi0, semi1)
    semds = (semd0, semd1)

    def _zrow(i, carry):
        for j in range(D // L):
            ebuf0[i, pl.ds(j * L, L)] = jnp.zeros((L,), jnp.float32)
        return carry

    lax.fori_loop(0, C, _zrow, 0)
    zbase = s * RPT
    for k in range(RPT // C):
        pltpu.sync_copy(ebuf0, aggr.at[pl.ds(zbase + k * C, C)])
    plsc.subcore_barrier()

    tbase = wid * EPT

    def _idx_dma(g, p):
        pltpu.async_copy(src_hbm.at[wid, g], sibufs[p], semis[p])

    def _issue(g, p, wait_scatter):
        if wait_scatter:
            pltpu.make_async_copy(ebufs[p], aggr.at[dbufs[p]], semss[p]).wait()
        pltpu.async_copy(dst_hbm.at[wid, g], dbufs[p], semds[p])
        pltpu.make_async_copy(src_hbm.at[wid, g], sibufs[p], semis[p]).wait()
        off = tbase + g * C
        pltpu.async_copy(e_hbm.at[pl.ds(off, C)], ebufs[p], semes[p])
        pltpu.async_copy(h_hbm.at[sibufs[p]], hbufs[p], semgs[p])

    def _compute(g, p, prefetch_idx):
        off = tbase + g * C
        pltpu.make_async_copy(e_hbm.at[pl.ds(off, C)], ebufs[p], semes[p]).wait()
        pltpu.make_async_copy(h_hbm.at[sibufs[p]], hbufs[p], semgs[p]).wait()
        if prefetch_idx:
            _idx_dma(g + 2, p)

        def _row(i, inner):
            for j in range(D // L):
                sl = pl.ds(j * L, L)
                ebufs[p][i, sl] = jnp.maximum(hbufs[p][i, sl] + ebufs[p][i, sl], 0.0)
            return inner

        lax.fori_loop(0, C, _row, 0)
        pltpu.make_async_copy(dst_hbm.at[wid, g], dbufs[p], semds[p]).wait()
        pltpu.async_copy(ebufs[p], aggr.at[dbufs[p]], semss[p], add=True)

    _idx_dma(0, 0)
    _idx_dma(1, 1)
    _issue(0, 0, False)
    _issue(1, 1, False)
    _compute(0, 0, True)
    _issue(2, 0, True)
    _compute(1, 1, True)

    def _pair(k, carry):
        g = 2 * k
        _issue(g + 1, 1, True)
        _compute(g, 0, True)
        _issue(g + 2, 0, True)
        _compute(g + 1, 1, True)
        return carry

    lax.fori_loop(1, NCHUNK // 2 - 1, _pair, 0)
    _issue(NCHUNK - 1, 1, True)
    _compute(NCHUNK - 2, 0, False)
    _compute(NCHUNK - 1, 1, False)
    pltpu.make_async_copy(ebuf0, aggr.at[dbuf0], sems0).wait()
    pltpu.make_async_copy(ebuf1, aggr.at[dbuf1], sems1).wait()
    plsc.subcore_barrier()

    pltpu.sync_copy(
        aggr.at[pl.ds(zbase, RPT)], out_hbm.at[c, pl.ds(zbase, RPT), :]
    )


_BN = 1000


def _apply_body(h_ref, p_ref, w_ref, b_ref, o_ref):
    x = (1.0 + EPS) * h_ref[...] + p_ref[0] + p_ref[1]
    o_ref[...] = jnp.maximum(
        jnp.dot(x, w_ref[...], preferred_element_type=jnp.float32) + b_ref[...], 0.0
    )


def _apply(h, partials, w, b):
    return pl.pallas_call(
        _apply_body,
        grid=(N // _BN,),
        in_specs=[
            pl.BlockSpec((_BN, D), lambda i: (i, 0)),
            pl.BlockSpec((NC, _BN, D), lambda i: (0, i, 0)),
            pl.BlockSpec((D, D), lambda i: (0, 0)),
            pl.BlockSpec((1, D), lambda i: (0, 0)),
        ],
        out_specs=pl.BlockSpec((_BN, D), lambda i: (i, 0)),
        out_shape=jax.ShapeDtypeStruct((N, D), jnp.float32),
    )(h, partials, w, b)


def kernel(node_feats, edge_feats, edge_index, We1, be1, We2, be2, W0, b0, W1, b1):
    src = edge_index[0].reshape(NW, NCHUNK, C)
    dst = edge_index[1].reshape(NW, NCHUNK, C)
    e = _edge_mlp(edge_feats, We1, be1.reshape(1, D), We2, be2.reshape(1, D))
    h = node_feats
    for w, b in ((W0, b0), (W1, b1)):
        partials = _sc_pass(e, h, src, dst)
        h = _apply(h, partials, w, b.reshape(1, D))
    return h
